# baseline (speedup 1.0000x reference)
import jax
import jax.numpy as jnp
from jax.experimental import pallas as pl
from jax.experimental.pallas import tpu as pltpu

_B = 4
_T = 14
_TP = _T + 1


def _zero_pads(ref, n_phase, c):
    def body(p, carry):
        ref[p, :, _T, :, :] = jnp.zeros((_B, _TP, c), jnp.float32)
        ref[p, :, :, _T, :] = jnp.zeros((_B, _TP, c), jnp.float32)
        return carry

    jax.lax.fori_loop(0, n_phase, body, 0)


def _conv_phase_split(in_ref, w_ref, s_in, out_ref):
    s_out = s_in // 2

    def one_phase(er, ec):
        acc = None
        for dy in range(3):
            pr, ar = (2 * er + dy) % s_in, (2 * er + dy) // s_in
            for dx in range(3):
                pc, ac = (2 * ec + dx) % s_in, (2 * ec + dx) // s_in
                sl = in_ref[pr * s_in + pc, :,
                            pl.ds(ar, _T), pl.ds(ac, _T), :]
                d = jax.lax.dot_general(
                    sl, w_ref[dy * 3 + dx],
                    (((3,), (0,)), ((), ())),
                    preferred_element_type=jnp.float32)
                acc = d if acc is None else acc + d
        return jnp.maximum(acc, 0.0)

    if out_ref is None:
        return one_phase(0, 0)

    def body(i, carry):
        er, ec = i // s_out, i % s_out
        out_ref[i, :, 0:_T, 0:_T, :] = one_phase(er, ec)
        return carry

    jax.lax.fori_loop(0, s_out * s_out, body, 0)
    return None


def _body(patch_ref, w1_ref, w2_ref, w3_ref, w4_ref, kT_ref, keys_ref,
          out_ref, s1, s2, s3):
    _zero_pads(s1, 64, 32)
    _zero_pads(s2, 16, 64)
    _zero_pads(s3, 4, 128)

    w1 = w1_ref[...]

    def l1_body(p, carry):
        sl = patch_ref[p].reshape(_B, _T, _T, 27)
        d = jax.lax.dot_general(sl, w1, (((3,), (0,)), ((), ())),
                                preferred_element_type=jnp.float32)
        s1[p, :, 0:_T, 0:_T, :] = jnp.maximum(d, 0.0)
        return carry

    jax.lax.fori_loop(0, 64, l1_body, 0)

    _conv_phase_split(s1, w2_ref, 8, s2)
    _conv_phase_split(s2, w3_ref, 4, s3)
    fea = _conv_phase_split(s3, w4_ref, 2, None)

    n2 = jnp.sum(fea * fea, axis=-1, keepdims=True)
    q = fea / jnp.maximum(jnp.sqrt(n2), 1e-12)

    kT = kT_ref[...]
    scores = jax.lax.dot_general(
        q, kT, (((3,), (0,)), ((), ())),
        preferred_element_type=jnp.float32)
    ksq = jnp.sum(kT * kT, axis=0)
    adj = scores - 0.5 * ksq
    m = jnp.max(adj, axis=-1, keepdims=True)
    ii = jax.lax.broadcasted_iota(jnp.int32, adj.shape, 3)
    cand = jnp.where(adj == m, ii, 512)
    nearest = jnp.min(cand, axis=-1, keepdims=True)
    onehot = (ii == nearest).astype(jnp.float32)
    nk = jax.lax.dot_general(
        onehot, keys_ref[...], (((3,), (0,)), ((), ())),
        preferred_element_type=jnp.float32)
    d = q - nk
    d2 = d * d
    out_ref[...] = jnp.sum(d2 * d2, axis=-1)


def kernel(x, W1, W2, W3, W4, keys):
    xh = jnp.transpose(x, (0, 2, 3, 1))
    xp = jnp.pad(xh, ((0, 0), (0, 2), (0, 2), (0, 0)))
    taps = []
    for dy in range(3):
        for dx in range(3):
            a = jax.lax.slice(xp, (0, dy, dx, 0), (_B, dy + 224, dx + 224, 3),
                              (1, 2, 2, 1))
            taps.append(a.reshape(_B, _T, 8, _T, 8, 3))
    pt = jnp.stack(taps, axis=5)
    patches = (pt.transpose(2, 4, 0, 1, 3, 5, 6)
               .reshape(64, _B, _T, _T * 27))

    w1 = jnp.transpose(W1, (2, 3, 1, 0)).reshape(27, 32)
    w2 = jnp.transpose(W2, (2, 3, 1, 0)).reshape(9, 32, 64)
    w3 = jnp.transpose(W3, (2, 3, 1, 0)).reshape(9, 64, 128)
    w4 = jnp.transpose(W4, (2, 3, 1, 0)).reshape(9, 128, 128)
    kT = keys.T

    out = pl.pallas_call(
        _body,
        out_shape=jax.ShapeDtypeStruct((_B, _T, _T), jnp.float32),
        scratch_shapes=[
            pltpu.VMEM((64, _B, _TP, _TP, 32), jnp.float32),
            pltpu.VMEM((16, _B, _TP, _TP, 64), jnp.float32),
            pltpu.VMEM((4, _B, _TP, _TP, 128), jnp.float32),
        ],
    )(patches, w1, w2, w3, w4, kT, keys)
    return out.reshape(_B, _T, _T, 1)
